# Initial kernel scaffold; baseline (speedup 1.0000x reference)
import functools

import jax
import jax.numpy as jnp
from jax import lax
from jax.experimental import pallas as pl
from jax.experimental.pallas import tpu as pltpu
from jax.experimental.pallas import tpu_sc as plsc

N = 10000
E = 320000
D = 128
EPS = 1e-5

NC = 2
NS = 16
NW = NC * NS
CHUNK = 128
NCHT = E // CHUNK
NCH_BASE = NCHT // NW
NCH_REM = NCHT - NCH_BASE * NW
RPT = N // NS


def _vmesh():
    return plsc.VectorSubcoreMesh(core_axis_name="c", subcore_axis_name="s",
                                  num_cores=NC, num_subcores=NS)


def _ln_rows(v, g, b):
    mu = jnp.mean(v, axis=-1, keepdims=True)
    d = v - mu
    var = jnp.mean(d * d, axis=-1, keepdims=True)
    return g * (d * lax.rsqrt(var + EPS)) + b


def _tc_qk(x, wqT, bq, wkT, bk):
    BN = 1250

    def body(x_r, wq_r, bq_r, wk_r, bk_r, q_r, k_r):
        xb = x_r[...]
        q_r[...] = jnp.dot(xb, wq_r[...], preferred_element_type=jnp.float32) + bq_r[...]
        k_r[...] = jnp.dot(xb, wk_r[...], preferred_element_type=jnp.float32) + bk_r[...]

    return pl.pallas_call(
        body,
        grid=(N // BN,),
        in_specs=[pl.BlockSpec((BN, D), lambda i: (i, 0)),
                  pl.BlockSpec((D, D), lambda i: (0, 0)),
                  pl.BlockSpec((1, D), lambda i: (0, 0)),
                  pl.BlockSpec((D, D), lambda i: (0, 0)),
                  pl.BlockSpec((1, D), lambda i: (0, 0))],
        out_specs=[pl.BlockSpec((BN, D), lambda i: (i, 0)),
                   pl.BlockSpec((BN, D), lambda i: (i, 0))],
        out_shape=[jax.ShapeDtypeStruct((N, D), jnp.float32)] * 2,
    )(x, wqT, bq, wkT, bk)


def _sc_gather(qall, kall, dst, src):
    @functools.partial(
        pl.kernel, mesh=_vmesh(),
        out_type=(jax.ShapeDtypeStruct((E, D), jnp.float32),
                  jax.ShapeDtypeStruct((E, D), jnp.float32)),
        scratch_types=[pltpu.VMEM((CHUNK,), jnp.int32),
                       pltpu.VMEM((CHUNK,), jnp.int32),
                       pltpu.VMEM((CHUNK, D), jnp.float32),
                       pltpu.VMEM((CHUNK, D), jnp.float32),
                       pltpu.SemaphoreType.DMA,
                       pltpu.SemaphoreType.DMA],
    )
    def k(q_hbm, k_hbm, dst_hbm, src_hbm, qd_hbm, ks_hbm,
          dst_v, src_v, rq_v, rk_v, s1, s2):
        wid = lax.axis_index("s") * NC + lax.axis_index("c")
        nch = NCH_BASE + jnp.where(wid < NCH_REM, 1, 0)

        def body(i, carry):
            base = (wid + NW * i) * CHUNK
            pltpu.sync_copy(dst_hbm.at[pl.ds(base, CHUNK)], dst_v)
            pltpu.sync_copy(src_hbm.at[pl.ds(base, CHUNK)], src_v)
            cq = pltpu.async_copy(q_hbm.at[dst_v], rq_v, s1)
            ck = pltpu.async_copy(k_hbm.at[src_v], rk_v, s2)
            cq.wait()
            ck.wait()
            pltpu.sync_copy(rq_v, qd_hbm.at[pl.ds(base, CHUNK)])
            pltpu.sync_copy(rk_v, ks_hbm.at[pl.ds(base, CHUNK)])
            return carry

        lax.fori_loop(0, nch, body, 0)

    return k(qall, kall, dst, src)


def _tc_edge(pc, qd, ks, weT, wcT, bc, ge, be):
    BE = 2000

    def body(pc_r, qd_r, ks_r, we_r, wc_r, bc_r, ge_r, be_r, conn_r, e_r):
        pcb = pc_r[...]
        eh = jnp.dot(pcb, we_r[...], preferred_element_type=jnp.float32)
        c1 = jnp.maximum(qd_r[...] + ks_r[...] + eh, 0.0)
        conn = jnp.dot(c1, wc_r[...], preferred_element_type=jnp.float32) + bc_r[...]
        conn_r[...] = conn
        e_r[...] = _ln_rows(pcb + conn, ge_r[...], be_r[...])

    return pl.pallas_call(
        body,
        grid=(E // BE,),
        in_specs=[pl.BlockSpec((BE, D), lambda i: (i, 0)),
                  pl.BlockSpec((BE, D), lambda i: (i, 0)),
                  pl.BlockSpec((BE, D), lambda i: (i, 0)),
                  pl.BlockSpec((D, D), lambda i: (0, 0)),
                  pl.BlockSpec((D, D), lambda i: (0, 0)),
                  pl.BlockSpec((1, D), lambda i: (0, 0)),
                  pl.BlockSpec((1, D), lambda i: (0, 0)),
                  pl.BlockSpec((1, D), lambda i: (0, 0))],
        out_specs=[pl.BlockSpec((BE, D), lambda i: (i, 0)),
                   pl.BlockSpec((BE, D), lambda i: (i, 0))],
        out_shape=[jax.ShapeDtypeStruct((E, D), jnp.float32)] * 2,
    )(pc, qd, ks, weT, wcT, bc, ge, be)


def _sc_scatter(conn, dst, zeros_nd):
    @functools.partial(
        pl.kernel, mesh=_vmesh(),
        out_type=jax.ShapeDtypeStruct((NC, N, D), jnp.float32),
        scratch_types=[pltpu.VMEM((CHUNK,), jnp.int32),
                       pltpu.VMEM((CHUNK, D), jnp.float32),
                       pltpu.VMEM_SHARED((N, D), jnp.float32)],
    )
    def k(conn_hbm, dst_hbm, zero_hbm, out_hbm, idx_v, rows_v, acc_sh):
        cid = lax.axis_index("c")
        sid = lax.axis_index("s")
        wid = sid * NC + cid
        nch = NCH_BASE + jnp.where(wid < NCH_REM, 1, 0)
        pltpu.sync_copy(zero_hbm.at[pl.ds(sid * RPT, RPT)],
                        acc_sh.at[pl.ds(sid * RPT, RPT)])
        plsc.subcore_barrier()

        def body(i, carry):
            base = (wid + NW * i) * CHUNK
            pltpu.sync_copy(dst_hbm.at[pl.ds(base, CHUNK)], idx_v)
            pltpu.sync_copy(conn_hbm.at[pl.ds(base, CHUNK)], rows_v)
            pltpu.sync_copy(rows_v, acc_sh.at[idx_v], add=True)
            return carry

        lax.fori_loop(0, nch, body, 0)
        plsc.subcore_barrier()
        pltpu.sync_copy(acc_sh.at[pl.ds(sid * RPT, RPT)],
                        out_hbm.at[cid, pl.ds(sid * RPT, RPT)])

    return k(conn, dst, zeros_nd)


def _tc_node(x, p0, p1, wnT, bn, gh, bh):
    BN = 1250

    def body(x_r, p0_r, p1_r, wn_r, bn_r, gh_r, bh_r, h_r):
        agg = p0_r[...] + p1_r[...]
        h = x_r[...] + jnp.dot(agg, wn_r[...], preferred_element_type=jnp.float32) + bn_r[...]
        h_r[...] = _ln_rows(h, gh_r[...], bh_r[...])

    return pl.pallas_call(
        body,
        grid=(N // BN,),
        in_specs=[pl.BlockSpec((BN, D), lambda i: (i, 0)),
                  pl.BlockSpec((BN, D), lambda i: (i, 0)),
                  pl.BlockSpec((BN, D), lambda i: (i, 0)),
                  pl.BlockSpec((D, D), lambda i: (0, 0)),
                  pl.BlockSpec((1, D), lambda i: (0, 0)),
                  pl.BlockSpec((1, D), lambda i: (0, 0)),
                  pl.BlockSpec((1, D), lambda i: (0, 0))],
        out_specs=pl.BlockSpec((BN, D), lambda i: (i, 0)),
        out_shape=jax.ShapeDtypeStruct((N, D), jnp.float32),
    )(x, p0, p1, wnT, bn, gh, bh)


def kernel(x, poly_conn, poly_index, WQ, bQ, WK, bK, WE, Wc, bc, Wn, bn,
           g_h, b_h, g_e, b_e):
    dst = poly_index[0]
    src = poly_index[1]
    qall, kall = _tc_qk(x, WQ.T, bQ[None], WK.T, bK[None])
    qd, ks = _sc_gather(qall, kall, dst, src)
    conn, e = _tc_edge(poly_conn, qd, ks, WE.T, Wc.T, bc[None],
                       g_e[None], b_e[None])
    parts = _sc_scatter(conn, dst, jnp.zeros((N, D), jnp.float32))
    h = _tc_node(x, parts[0], parts[1], Wn.T, bn[None], g_h[None], b_h[None])
    return (h, e)


# speedup vs baseline: 3.3181x; 3.3181x over previous
import functools

import jax
import jax.numpy as jnp
from jax import lax
from jax.experimental import pallas as pl
from jax.experimental.pallas import tpu as pltpu
from jax.experimental.pallas import tpu_sc as plsc

N = 10000
E = 320000
D = 128
EPS = 1e-5

NC = 2
NS = 16
NW = NC * NS
CHUNK = 128
NCHT = E // CHUNK
NCH_BASE = NCHT // NW
NCH_REM = NCHT - NCH_BASE * NW
N_PAD = 10240
RPT = N_PAD // NS


def _vmesh():
    return plsc.VectorSubcoreMesh(core_axis_name="c", subcore_axis_name="s",
                                  num_cores=NC, num_subcores=NS)


def _ln_rows(v, g, b):
    mu = jnp.mean(v, axis=-1, keepdims=True)
    d = v - mu
    var = jnp.mean(d * d, axis=-1, keepdims=True)
    return g * (d * lax.rsqrt(var + EPS)) + b


def _tc_qk(x, wqT, bq, wkT, bk):
    BN = 2000

    def body(x_r, wq_r, bq_r, wk_r, bk_r, q_r, k_r):
        xb = x_r[...]
        q_r[...] = jnp.dot(xb, wq_r[...], preferred_element_type=jnp.float32) + bq_r[...]
        k_r[...] = jnp.dot(xb, wk_r[...], preferred_element_type=jnp.float32) + bk_r[...]

    return pl.pallas_call(
        body,
        grid=(N // BN,),
        in_specs=[pl.BlockSpec((BN, D), lambda i: (i, 0)),
                  pl.BlockSpec((D, D), lambda i: (0, 0)),
                  pl.BlockSpec((1, D), lambda i: (0, 0)),
                  pl.BlockSpec((D, D), lambda i: (0, 0)),
                  pl.BlockSpec((1, D), lambda i: (0, 0))],
        out_specs=[pl.BlockSpec((BN, D), lambda i: (i, 0)),
                   pl.BlockSpec((BN, D), lambda i: (i, 0))],
        out_shape=[jax.ShapeDtypeStruct((N, D), jnp.float32)] * 2,
    )(x, wqT, bq, wkT, bk)


def _sc_gather(qall, kall, dst, src):
    @functools.partial(
        pl.kernel, mesh=_vmesh(),
        out_type=(jax.ShapeDtypeStruct((E, D), jnp.float32),
                  jax.ShapeDtypeStruct((E, D), jnp.float32)),
        scratch_types=[pltpu.VMEM((CHUNK,), jnp.int32),
                       pltpu.VMEM((CHUNK,), jnp.int32),
                       pltpu.VMEM((CHUNK, D), jnp.float32),
                       pltpu.VMEM((CHUNK, D), jnp.float32),
                       pltpu.SemaphoreType.DMA,
                       pltpu.SemaphoreType.DMA],
    )
    def k(q_hbm, k_hbm, dst_hbm, src_hbm, qd_hbm, ks_hbm,
          dst_v, src_v, rq_v, rk_v, s1, s2):
        wid = lax.axis_index("s") * NC + lax.axis_index("c")
        nch = NCH_BASE + jnp.where(wid < NCH_REM, 1, 0)

        def body(i, carry):
            base = (wid + NW * i) * CHUNK
            pltpu.sync_copy(dst_hbm.at[pl.ds(base, CHUNK)], dst_v)
            pltpu.sync_copy(src_hbm.at[pl.ds(base, CHUNK)], src_v)
            cq = pltpu.async_copy(q_hbm.at[dst_v], rq_v, s1)
            ck = pltpu.async_copy(k_hbm.at[src_v], rk_v, s2)
            cq.wait()
            ck.wait()
            pltpu.sync_copy(rq_v, qd_hbm.at[pl.ds(base, CHUNK)])
            pltpu.sync_copy(rk_v, ks_hbm.at[pl.ds(base, CHUNK)])
            return carry

        lax.fori_loop(0, nch, body, 0)

    return k(qall, kall, dst, src)


def _tc_edge(pc, qd, ks, weT, wcT, bc, ge, be):
    BE = 2000

    def body(pc_r, qd_r, ks_r, we_r, wc_r, bc_r, ge_r, be_r, conn_r, e_r):
        pcb = pc_r[...]
        eh = jnp.dot(pcb, we_r[...], preferred_element_type=jnp.float32)
        c1 = jnp.maximum(qd_r[...] + ks_r[...] + eh, 0.0)
        conn = jnp.dot(c1, wc_r[...], preferred_element_type=jnp.float32) + bc_r[...]
        conn_r[...] = conn
        e_r[...] = _ln_rows(pcb + conn, ge_r[...], be_r[...])

    return pl.pallas_call(
        body,
        grid=(E // BE,),
        in_specs=[pl.BlockSpec((BE, D), lambda i: (i, 0)),
                  pl.BlockSpec((BE, D), lambda i: (i, 0)),
                  pl.BlockSpec((BE, D), lambda i: (i, 0)),
                  pl.BlockSpec((D, D), lambda i: (0, 0)),
                  pl.BlockSpec((D, D), lambda i: (0, 0)),
                  pl.BlockSpec((1, D), lambda i: (0, 0)),
                  pl.BlockSpec((1, D), lambda i: (0, 0)),
                  pl.BlockSpec((1, D), lambda i: (0, 0))],
        out_specs=[pl.BlockSpec((BE, D), lambda i: (i, 0)),
                   pl.BlockSpec((BE, D), lambda i: (i, 0))],
        out_shape=[jax.ShapeDtypeStruct((E, D), jnp.float32)] * 2,
    )(pc, qd, ks, weT, wcT, bc, ge, be)


def _sc_scatter(conn, dst, zeros_nd):
    @functools.partial(
        pl.kernel, mesh=_vmesh(),
        out_type=jax.ShapeDtypeStruct((NC, N_PAD, D), jnp.float32),
        scratch_types=[pltpu.VMEM((CHUNK,), jnp.int32),
                       pltpu.VMEM((CHUNK, D), jnp.float32),
                       pltpu.VMEM_SHARED((N_PAD, D), jnp.float32)],
    )
    def k(conn_hbm, dst_hbm, zero_hbm, out_hbm, idx_v, rows_v, acc_sh):
        cid = lax.axis_index("c")
        sid = lax.axis_index("s")
        wid = sid * NC + cid
        nch = NCH_BASE + jnp.where(wid < NCH_REM, 1, 0)
        pltpu.sync_copy(zero_hbm.at[pl.ds(sid * RPT, RPT)],
                        acc_sh.at[pl.ds(sid * RPT, RPT)])
        plsc.subcore_barrier()

        def body(i, carry):
            base = (wid + NW * i) * CHUNK
            pltpu.sync_copy(dst_hbm.at[pl.ds(base, CHUNK)], idx_v)
            pltpu.sync_copy(conn_hbm.at[pl.ds(base, CHUNK)], rows_v)
            pltpu.sync_copy(rows_v, acc_sh.at[idx_v], add=True)
            return carry

        lax.fori_loop(0, nch, body, 0)
        plsc.subcore_barrier()
        pltpu.sync_copy(acc_sh.at[pl.ds(sid * RPT, RPT)],
                        out_hbm.at[cid, pl.ds(sid * RPT, RPT)])

    return k(conn, dst, zeros_nd)


def _tc_node(x, parts, wnT, bn, gh, bh):
    BN = 2000

    def body(x_r, p_r, wn_r, bn_r, gh_r, bh_r, h_r):
        agg = p_r[0] + p_r[1]
        h = x_r[...] + jnp.dot(agg, wn_r[...], preferred_element_type=jnp.float32) + bn_r[...]
        h_r[...] = _ln_rows(h, gh_r[...], bh_r[...])

    return pl.pallas_call(
        body,
        grid=(N // BN,),
        in_specs=[pl.BlockSpec((BN, D), lambda i: (i, 0)),
                  pl.BlockSpec((NC, BN, D), lambda i: (0, i, 0)),
                  pl.BlockSpec((D, D), lambda i: (0, 0)),
                  pl.BlockSpec((1, D), lambda i: (0, 0)),
                  pl.BlockSpec((1, D), lambda i: (0, 0)),
                  pl.BlockSpec((1, D), lambda i: (0, 0))],
        out_specs=pl.BlockSpec((BN, D), lambda i: (i, 0)),
        out_shape=jax.ShapeDtypeStruct((N, D), jnp.float32),
    )(x, parts, wnT, bn, gh, bh)


def kernel(x, poly_conn, poly_index, WQ, bQ, WK, bK, WE, Wc, bc, Wn, bn,
           g_h, b_h, g_e, b_e):
    dst = poly_index[0]
    src = poly_index[1]
    qall, kall = _tc_qk(x, WQ.T, bQ[None], WK.T, bK[None])
    qd, ks = _sc_gather(qall, kall, dst, src)
    conn, e = _tc_edge(poly_conn, qd, ks, WE.T, Wc.T, bc[None],
                       g_e[None], b_e[None])
    parts = _sc_scatter(conn, dst, jnp.zeros((N_PAD, D), jnp.float32))
    h = _tc_node(x, parts, Wn.T, bn[None], g_h[None], b_h[None])
    return (h, e)


# speedup vs baseline: 4.1207x; 1.2419x over previous
import functools

import jax
import jax.numpy as jnp
from jax import lax
from jax.experimental import pallas as pl
from jax.experimental.pallas import tpu as pltpu
from jax.experimental.pallas import tpu_sc as plsc

N = 10000
E = 320000
D = 128
EPS = 1e-5

NC = 2
NS = 16
NW = NC * NS
CHUNK = 128
NCHT = E // CHUNK
NCH_BASE = NCHT // NW
NCH_REM = NCHT - NCH_BASE * NW
N_PAD = 10240
RPT = N_PAD // NS


def _vmesh():
    return plsc.VectorSubcoreMesh(core_axis_name="c", subcore_axis_name="s",
                                  num_cores=NC, num_subcores=NS)


def _ln_rows(v, g, b):
    mu = jnp.mean(v, axis=-1, keepdims=True)
    d = v - mu
    var = jnp.mean(d * d, axis=-1, keepdims=True)
    return g * (d * lax.rsqrt(var + EPS)) + b


def _tc_qk(x, wqT, bq, wkT, bk):
    BN = 2000

    def body(x_r, wq_r, bq_r, wk_r, bk_r, q_r, k_r):
        xb = x_r[...]
        q_r[...] = jnp.dot(xb, wq_r[...], preferred_element_type=jnp.float32) + bq_r[...]
        k_r[...] = jnp.dot(xb, wk_r[...], preferred_element_type=jnp.float32) + bk_r[...]

    return pl.pallas_call(
        body,
        grid=(N // BN,),
        in_specs=[pl.BlockSpec((BN, D), lambda i: (i, 0)),
                  pl.BlockSpec((D, D), lambda i: (0, 0)),
                  pl.BlockSpec((1, D), lambda i: (0, 0)),
                  pl.BlockSpec((D, D), lambda i: (0, 0)),
                  pl.BlockSpec((1, D), lambda i: (0, 0))],
        out_specs=[pl.BlockSpec((BN, D), lambda i: (i, 0)),
                   pl.BlockSpec((BN, D), lambda i: (i, 0))],
        out_shape=[jax.ShapeDtypeStruct((N, D), jnp.float32)] * 2,
    )(x, wqT, bq, wkT, bk)


EPW = E // NW
GCH = EPW // CHUNK
NSETS = 3
GP = GCH // NSETS
GREM = EPW - GCH * CHUNK


def _sc_gather(qall, kall, dst, src):
    @functools.partial(
        pl.kernel, mesh=_vmesh(),
        out_type=(jax.ShapeDtypeStruct((E, D), jnp.float32),
                  jax.ShapeDtypeStruct((E, D), jnp.float32)),
        scratch_types=[pltpu.VMEM((EPW,), jnp.int32),
                       pltpu.VMEM((EPW,), jnp.int32)]
                      + [pltpu.VMEM((CHUNK, D), jnp.float32)] * (2 * NSETS)
                      + [pltpu.SemaphoreType.DMA] * (2 * NSETS),
    )
    def k(q_hbm, k_hbm, dst_hbm, src_hbm, qd_hbm, ks_hbm, dsti, srci, *bufs):
        rqs = bufs[0:NSETS]
        rks = bufs[NSETS:2 * NSETS]
        gsems = bufs[2 * NSETS:3 * NSETS]
        ssems = bufs[3 * NSETS:4 * NSETS]
        wid = lax.axis_index("s") * NC + lax.axis_index("c")
        eb = wid * EPW
        pltpu.sync_copy(dst_hbm.at[pl.ds(eb, EPW)], dsti)
        pltpu.sync_copy(src_hbm.at[pl.ds(eb, EPW)], srci)

        def issue_gather(c, s):
            pltpu.async_copy(q_hbm.at[dsti.at[pl.ds(c * CHUNK, CHUNK)]],
                             rqs[s], gsems[s])
            pltpu.async_copy(k_hbm.at[srci.at[pl.ds(c * CHUNK, CHUNK)]],
                             rks[s], gsems[s])

        def wait_gather(s):
            pltpu.make_async_copy(q_hbm.at[dsti.at[pl.ds(0, CHUNK)]],
                                  rqs[s], gsems[s]).wait()
            pltpu.make_async_copy(k_hbm.at[srci.at[pl.ds(0, CHUNK)]],
                                  rks[s], gsems[s]).wait()

        def issue_store(c, s):
            pltpu.async_copy(rqs[s], qd_hbm.at[pl.ds(eb + c * CHUNK, CHUNK)],
                             ssems[s])
            pltpu.async_copy(rks[s], ks_hbm.at[pl.ds(eb + c * CHUNK, CHUNK)],
                             ssems[s])

        def wait_store(s):
            pltpu.make_async_copy(rqs[s], qd_hbm.at[pl.ds(eb, CHUNK)],
                                  ssems[s]).wait()
            pltpu.make_async_copy(rks[s], ks_hbm.at[pl.ds(eb, CHUNK)],
                                  ssems[s]).wait()

        for s in range(NSETS):
            issue_gather(s, s)

        def body(j, carry):
            c = NSETS * j
            for s in range(NSETS):
                wait_gather(s)
                issue_store(c + s, s)
            for s in range(NSETS):
                wait_store(s)

                @pl.when(c + s + NSETS < GCH)
                def _():
                    issue_gather(c + s + NSETS, s)

            return carry

        lax.fori_loop(0, GP, body, 0)

        pltpu.async_copy(q_hbm.at[dsti.at[pl.ds(GCH * CHUNK, GREM)]],
                         rqs[0].at[pl.ds(0, GREM)], gsems[0])
        pltpu.async_copy(k_hbm.at[srci.at[pl.ds(GCH * CHUNK, GREM)]],
                         rks[0].at[pl.ds(0, GREM)], gsems[0])
        pltpu.make_async_copy(q_hbm.at[dsti.at[pl.ds(0, GREM)]],
                              rqs[0].at[pl.ds(0, GREM)], gsems[0]).wait()
        pltpu.make_async_copy(k_hbm.at[srci.at[pl.ds(0, GREM)]],
                              rks[0].at[pl.ds(0, GREM)], gsems[0]).wait()
        pltpu.sync_copy(rqs[0].at[pl.ds(0, GREM)],
                        qd_hbm.at[pl.ds(eb + GCH * CHUNK, GREM)])
        pltpu.sync_copy(rks[0].at[pl.ds(0, GREM)],
                        ks_hbm.at[pl.ds(eb + GCH * CHUNK, GREM)])

    return k(qall, kall, dst, src)


def _tc_edge(pc, qd, ks, weT, wcT, bc, ge, be):
    BE = 2000

    def body(pc_r, qd_r, ks_r, we_r, wc_r, bc_r, ge_r, be_r, conn_r, e_r):
        pcb = pc_r[...]
        eh = jnp.dot(pcb, we_r[...], preferred_element_type=jnp.float32)
        c1 = jnp.maximum(qd_r[...] + ks_r[...] + eh, 0.0)
        conn = jnp.dot(c1, wc_r[...], preferred_element_type=jnp.float32) + bc_r[...]
        conn_r[...] = conn
        e_r[...] = _ln_rows(pcb + conn, ge_r[...], be_r[...])

    return pl.pallas_call(
        body,
        grid=(E // BE,),
        in_specs=[pl.BlockSpec((BE, D), lambda i: (i, 0)),
                  pl.BlockSpec((BE, D), lambda i: (i, 0)),
                  pl.BlockSpec((BE, D), lambda i: (i, 0)),
                  pl.BlockSpec((D, D), lambda i: (0, 0)),
                  pl.BlockSpec((D, D), lambda i: (0, 0)),
                  pl.BlockSpec((1, D), lambda i: (0, 0)),
                  pl.BlockSpec((1, D), lambda i: (0, 0)),
                  pl.BlockSpec((1, D), lambda i: (0, 0))],
        out_specs=[pl.BlockSpec((BE, D), lambda i: (i, 0)),
                   pl.BlockSpec((BE, D), lambda i: (i, 0))],
        out_shape=[jax.ShapeDtypeStruct((E, D), jnp.float32)] * 2,
    )(pc, qd, ks, weT, wcT, bc, ge, be)


SSETS = 2
SGP = GCH // SSETS


def _sc_scatter(conn, dst, zeros_nd):
    @functools.partial(
        pl.kernel, mesh=_vmesh(),
        out_type=jax.ShapeDtypeStruct((NC, N_PAD, D), jnp.float32),
        scratch_types=[pltpu.VMEM_SHARED((N_PAD, D), jnp.float32)]
                      + [pltpu.VMEM((CHUNK,), jnp.int32)] * SSETS
                      + [pltpu.VMEM((CHUNK, D), jnp.float32)] * SSETS
                      + [pltpu.SemaphoreType.DMA] * (2 * SSETS),
    )
    def k(conn_hbm, dst_hbm, zero_hbm, out_hbm, acc_sh, *bufs):
        idxs = bufs[0:SSETS]
        rows = bufs[SSETS:2 * SSETS]
        lsems = bufs[2 * SSETS:3 * SSETS]
        asems = bufs[3 * SSETS:4 * SSETS]
        cid = lax.axis_index("c")
        sid = lax.axis_index("s")
        wid = sid * NC + cid
        nch = NCH_BASE + jnp.where(wid < NCH_REM, 1, 0)
        pltpu.sync_copy(zero_hbm.at[pl.ds(sid * RPT, RPT)],
                        acc_sh.at[pl.ds(sid * RPT, RPT)])
        plsc.subcore_barrier()

        def issue_load(c, s):
            base = (wid + NW * c) * CHUNK
            pltpu.async_copy(dst_hbm.at[pl.ds(base, CHUNK)], idxs[s], lsems[s])
            pltpu.async_copy(conn_hbm.at[pl.ds(base, CHUNK)], rows[s], lsems[s])

        def wait_load(s):
            pltpu.make_async_copy(dst_hbm.at[pl.ds(0, CHUNK)],
                                  idxs[s], lsems[s]).wait()
            pltpu.make_async_copy(conn_hbm.at[pl.ds(0, CHUNK)],
                                  rows[s], lsems[s]).wait()

        def issue_scatter(s):
            pltpu.async_copy(rows[s], acc_sh.at[idxs[s]], asems[s], add=True)

        def wait_scatter(s):
            pltpu.make_async_copy(rows[s], acc_sh.at[idxs[s]], asems[s]).wait()

        for s in range(SSETS):
            issue_load(s, s)

        def body(j, carry):
            c = SSETS * j
            for s in range(SSETS):
                wait_load(s)
                issue_scatter(s)
            for s in range(SSETS):
                wait_scatter(s)

                @pl.when(c + s + SSETS < nch)
                def _():
                    issue_load(c + s + SSETS, s)

            return carry

        lax.fori_loop(0, SGP, body, 0)

        @pl.when(nch > SSETS * SGP)
        def _():
            wait_load(0)
            issue_scatter(0)
            wait_scatter(0)

        plsc.subcore_barrier()
        pltpu.sync_copy(acc_sh.at[pl.ds(sid * RPT, RPT)],
                        out_hbm.at[cid, pl.ds(sid * RPT, RPT)])

    return k(conn, dst, zeros_nd)


def _tc_node(x, parts, wnT, bn, gh, bh):
    BN = 2000

    def body(x_r, p_r, wn_r, bn_r, gh_r, bh_r, h_r):
        agg = p_r[0] + p_r[1]
        h = x_r[...] + jnp.dot(agg, wn_r[...], preferred_element_type=jnp.float32) + bn_r[...]
        h_r[...] = _ln_rows(h, gh_r[...], bh_r[...])

    return pl.pallas_call(
        body,
        grid=(N // BN,),
        in_specs=[pl.BlockSpec((BN, D), lambda i: (i, 0)),
                  pl.BlockSpec((NC, BN, D), lambda i: (0, i, 0)),
                  pl.BlockSpec((D, D), lambda i: (0, 0)),
                  pl.BlockSpec((1, D), lambda i: (0, 0)),
                  pl.BlockSpec((1, D), lambda i: (0, 0)),
                  pl.BlockSpec((1, D), lambda i: (0, 0))],
        out_specs=pl.BlockSpec((BN, D), lambda i: (i, 0)),
        out_shape=jax.ShapeDtypeStruct((N, D), jnp.float32),
    )(x, parts, wnT, bn, gh, bh)


def kernel(x, poly_conn, poly_index, WQ, bQ, WK, bK, WE, Wc, bc, Wn, bn,
           g_h, b_h, g_e, b_e):
    dst = poly_index[0]
    src = poly_index[1]
    qall, kall = _tc_qk(x, WQ.T, bQ[None], WK.T, bK[None])
    qd, ks = _sc_gather(qall, kall, dst, src)
    conn, e = _tc_edge(poly_conn, qd, ks, WE.T, Wc.T, bc[None],
                       g_e[None], b_e[None])
    parts = _sc_scatter(conn, dst, jnp.zeros((N_PAD, D), jnp.float32))
    h = _tc_node(x, parts, Wn.T, bn[None], g_h[None], b_h[None])
    return (h, e)


# speedup vs baseline: 4.4570x; 1.0816x over previous
import functools

import jax
import jax.numpy as jnp
from jax import lax
from jax.experimental import pallas as pl
from jax.experimental.pallas import tpu as pltpu
from jax.experimental.pallas import tpu_sc as plsc

N = 10000
E = 320000
D = 128
EPS = 1e-5

NC = 2
NS = 16
NW = NC * NS
CHUNK = 128
NHALF = 2
EH = E // NHALF
EPW = EH // NW
GCH = EPW // CHUNK
GREM = EPW - GCH * CHUNK
GSETS = 3
GP = GCH // GSETS
SSETS = 2
SGP = GCH // SSETS
N_PAD = 10240
RPT = N_PAD // NS


def _vmesh():
    return plsc.VectorSubcoreMesh(core_axis_name="c", subcore_axis_name="s",
                                  num_cores=NC, num_subcores=NS)


def _ln_rows(v, g, b):
    mu = jnp.mean(v, axis=-1, keepdims=True)
    d = v - mu
    var = jnp.mean(d * d, axis=-1, keepdims=True)
    return g * (d * lax.rsqrt(var + EPS)) + b


def _tc_qk(x, wqT, bq, wkT, bk):
    BN = 2000

    def body(x_r, wq_r, bq_r, wk_r, bk_r, q_r, k_r):
        xb = x_r[...]
        q_r[...] = jnp.dot(xb, wq_r[...], preferred_element_type=jnp.float32) + bq_r[...]
        k_r[...] = jnp.dot(xb, wk_r[...], preferred_element_type=jnp.float32) + bk_r[...]

    return pl.pallas_call(
        body,
        grid=(N // BN,),
        in_specs=[pl.BlockSpec((BN, D), lambda i: (i, 0)),
                  pl.BlockSpec((D, D), lambda i: (0, 0)),
                  pl.BlockSpec((1, D), lambda i: (0, 0)),
                  pl.BlockSpec((D, D), lambda i: (0, 0)),
                  pl.BlockSpec((1, D), lambda i: (0, 0))],
        out_specs=[pl.BlockSpec((BN, D), lambda i: (i, 0)),
                   pl.BlockSpec((BN, D), lambda i: (i, 0))],
        out_shape=[jax.ShapeDtypeStruct((N, D), jnp.float32)] * 2,
    )(x, wqT, bq, wkT, bk)


def _sc_gather(qall, kall, dst_h, src_h):
    @functools.partial(
        pl.kernel, mesh=_vmesh(),
        out_type=(jax.ShapeDtypeStruct((EH, D), jnp.float32),
                  jax.ShapeDtypeStruct((EH, D), jnp.float32)),
        scratch_types=[pltpu.VMEM((EPW,), jnp.int32),
                       pltpu.VMEM((EPW,), jnp.int32)]
                      + [pltpu.VMEM((CHUNK, D), jnp.float32)] * (2 * GSETS)
                      + [pltpu.SemaphoreType.DMA] * (2 * GSETS),
    )
    def k(q_hbm, k_hbm, dst_hbm, src_hbm, qd_hbm, ks_hbm, dsti, srci, *bufs):
        rqs = bufs[0:GSETS]
        rks = bufs[GSETS:2 * GSETS]
        gsems = bufs[2 * GSETS:3 * GSETS]
        ssems = bufs[3 * GSETS:4 * GSETS]
        wid = lax.axis_index("s") * NC + lax.axis_index("c")
        eb = wid * EPW
        pltpu.sync_copy(dst_hbm.at[pl.ds(eb, EPW)], dsti)
        pltpu.sync_copy(src_hbm.at[pl.ds(eb, EPW)], srci)

        def issue_gather(c, s):
            pltpu.async_copy(q_hbm.at[dsti.at[pl.ds(c * CHUNK, CHUNK)]],
                             rqs[s], gsems[s])
            pltpu.async_copy(k_hbm.at[srci.at[pl.ds(c * CHUNK, CHUNK)]],
                             rks[s], gsems[s])

        def wait_gather(s):
            pltpu.make_async_copy(q_hbm.at[dsti.at[pl.ds(0, CHUNK)]],
                                  rqs[s], gsems[s]).wait()
            pltpu.make_async_copy(k_hbm.at[srci.at[pl.ds(0, CHUNK)]],
                                  rks[s], gsems[s]).wait()

        def issue_store(c, s):
            pltpu.async_copy(rqs[s], qd_hbm.at[pl.ds(eb + c * CHUNK, CHUNK)],
                             ssems[s])
            pltpu.async_copy(rks[s], ks_hbm.at[pl.ds(eb + c * CHUNK, CHUNK)],
                             ssems[s])

        def wait_store(s):
            pltpu.make_async_copy(rqs[s], qd_hbm.at[pl.ds(eb, CHUNK)],
                                  ssems[s]).wait()
            pltpu.make_async_copy(rks[s], ks_hbm.at[pl.ds(eb, CHUNK)],
                                  ssems[s]).wait()

        for s in range(GSETS):
            issue_gather(s, s)

        def body(j, carry):
            c = GSETS * j
            for s in range(GSETS):
                wait_gather(s)
                issue_store(c + s, s)
            for s in range(GSETS):
                wait_store(s)

                @pl.when(c + s + GSETS < GCH)
                def _():
                    issue_gather(c + s + GSETS, s)

            return carry

        lax.fori_loop(0, GP, body, 0)

        pltpu.async_copy(q_hbm.at[dsti.at[pl.ds(GCH * CHUNK, GREM)]],
                         rqs[0].at[pl.ds(0, GREM)], gsems[0])
        pltpu.async_copy(k_hbm.at[srci.at[pl.ds(GCH * CHUNK, GREM)]],
                         rks[0].at[pl.ds(0, GREM)], gsems[0])
        pltpu.make_async_copy(q_hbm.at[dsti.at[pl.ds(0, GREM)]],
                              rqs[0].at[pl.ds(0, GREM)], gsems[0]).wait()
        pltpu.make_async_copy(k_hbm.at[srci.at[pl.ds(0, GREM)]],
                              rks[0].at[pl.ds(0, GREM)], gsems[0]).wait()
        pltpu.sync_copy(rqs[0].at[pl.ds(0, GREM)],
                        qd_hbm.at[pl.ds(eb + GCH * CHUNK, GREM)])
        pltpu.sync_copy(rks[0].at[pl.ds(0, GREM)],
                        ks_hbm.at[pl.ds(eb + GCH * CHUNK, GREM)])

    return k(qall, kall, dst_h, src_h)


def _tc_edge(pc, qd, ks, weT, wcT, bc, ge, be, half, e_prev=None):
    BE = 2000
    hoff = half * (EH // BE)

    def body(pc_r, qd_r, ks_r, we_r, wc_r, bc_r, ge_r, be_r, *rest):
        conn_r, e_r = rest[-2], rest[-1]
        pcb = pc_r[...]
        eh = jnp.dot(pcb, we_r[...], preferred_element_type=jnp.float32)
        c1 = jnp.maximum(qd_r[...] + ks_r[...] + eh, 0.0)
        conn = jnp.dot(c1, wc_r[...], preferred_element_type=jnp.float32) + bc_r[...]
        conn_r[...] = conn
        e_r[...] = _ln_rows(pcb + conn, ge_r[...], be_r[...])

    in_specs = [pl.BlockSpec((BE, D), lambda i: (i + hoff, 0)),
                pl.BlockSpec((BE, D), lambda i: (i, 0)),
                pl.BlockSpec((BE, D), lambda i: (i, 0)),
                pl.BlockSpec((D, D), lambda i: (0, 0)),
                pl.BlockSpec((D, D), lambda i: (0, 0)),
                pl.BlockSpec((1, D), lambda i: (0, 0)),
                pl.BlockSpec((1, D), lambda i: (0, 0)),
                pl.BlockSpec((1, D), lambda i: (0, 0))]
    args = (pc, qd, ks, weT, wcT, bc, ge, be)
    aliases = {}
    if e_prev is not None:
        in_specs.append(pl.BlockSpec(memory_space=pl.ANY))
        args = args + (e_prev,)
        aliases = {8: 1}

    return pl.pallas_call(
        body,
        grid=(EH // BE,),
        in_specs=in_specs,
        out_specs=[pl.BlockSpec((BE, D), lambda i: (i, 0)),
                   pl.BlockSpec((BE, D), lambda i: (i + hoff, 0))],
        out_shape=[jax.ShapeDtypeStruct((EH, D), jnp.float32),
                   jax.ShapeDtypeStruct((E, D), jnp.float32)],
        input_output_aliases=aliases,
    )(*args)


def _sc_scatter(conn_h, dst_h, zeros_nd):
    @functools.partial(
        pl.kernel, mesh=_vmesh(),
        out_type=jax.ShapeDtypeStruct((NC, N_PAD, D), jnp.float32),
        scratch_types=[pltpu.VMEM_SHARED((N_PAD, D), jnp.float32),
                       pltpu.VMEM((GREM,), jnp.int32)]
                      + [pltpu.VMEM((CHUNK,), jnp.int32)] * SSETS
                      + [pltpu.VMEM((CHUNK, D), jnp.float32)] * SSETS
                      + [pltpu.SemaphoreType.DMA] * (2 * SSETS),
    )
    def k(conn_hbm, dst_hbm, zero_hbm, out_hbm, acc_sh, idx_rem, *bufs):
        idxs = bufs[0:SSETS]
        rows = bufs[SSETS:2 * SSETS]
        lsems = bufs[2 * SSETS:3 * SSETS]
        asems = bufs[3 * SSETS:4 * SSETS]
        cid = lax.axis_index("c")
        sid = lax.axis_index("s")
        wid = sid * NC + cid
        eb = wid * EPW
        pltpu.sync_copy(zero_hbm.at[pl.ds(sid * RPT, RPT)],
                        acc_sh.at[pl.ds(sid * RPT, RPT)])
        plsc.subcore_barrier()

        def issue_load(c, s):
            base = eb + c * CHUNK
            pltpu.async_copy(dst_hbm.at[pl.ds(base, CHUNK)], idxs[s], lsems[s])
            pltpu.async_copy(conn_hbm.at[pl.ds(base, CHUNK)], rows[s], lsems[s])

        def wait_load(s):
            pltpu.make_async_copy(dst_hbm.at[pl.ds(0, CHUNK)],
                                  idxs[s], lsems[s]).wait()
            pltpu.make_async_copy(conn_hbm.at[pl.ds(0, CHUNK)],
                                  rows[s], lsems[s]).wait()

        def issue_scatter(s):
            pltpu.async_copy(rows[s], acc_sh.at[idxs[s]], asems[s], add=True)

        def wait_scatter(s):
            pltpu.make_async_copy(rows[s], acc_sh.at[idxs[s]], asems[s]).wait()

        for s in range(SSETS):
            issue_load(s, s)

        def body(j, carry):
            c = SSETS * j
            for s in range(SSETS):
                wait_load(s)
                issue_scatter(s)
            for s in range(SSETS):
                wait_scatter(s)

                @pl.when(c + s + SSETS < GCH)
                def _():
                    issue_load(c + s + SSETS, s)

            return carry

        lax.fori_loop(0, SGP, body, 0)

        wait_load(0)
        issue_scatter(0)
        wait_scatter(0)

        rbase = eb + GCH * CHUNK
        pltpu.sync_copy(dst_hbm.at[pl.ds(rbase, GREM)], idx_rem)
        pltpu.sync_copy(conn_hbm.at[pl.ds(rbase, GREM)],
                        rows[1].at[pl.ds(0, GREM)])
        pltpu.sync_copy(rows[1].at[pl.ds(0, GREM)], acc_sh.at[idx_rem],
                        add=True)

        plsc.subcore_barrier()
        pltpu.sync_copy(acc_sh.at[pl.ds(sid * RPT, RPT)],
                        out_hbm.at[cid, pl.ds(sid * RPT, RPT)])

    return k(conn_h, dst_h, zeros_nd)


def _tc_node(x, parts_a, parts_b, wnT, bn, gh, bh):
    BN = 2000

    def body(x_r, pa_r, pb_r, wn_r, bn_r, gh_r, bh_r, h_r):
        agg = pa_r[0] + pa_r[1] + pb_r[0] + pb_r[1]
        h = x_r[...] + jnp.dot(agg, wn_r[...], preferred_element_type=jnp.float32) + bn_r[...]
        h_r[...] = _ln_rows(h, gh_r[...], bh_r[...])

    return pl.pallas_call(
        body,
        grid=(N // BN,),
        in_specs=[pl.BlockSpec((BN, D), lambda i: (i, 0)),
                  pl.BlockSpec((NC, BN, D), lambda i: (0, i, 0)),
                  pl.BlockSpec((NC, BN, D), lambda i: (0, i, 0)),
                  pl.BlockSpec((D, D), lambda i: (0, 0)),
                  pl.BlockSpec((1, D), lambda i: (0, 0)),
                  pl.BlockSpec((1, D), lambda i: (0, 0)),
                  pl.BlockSpec((1, D), lambda i: (0, 0))],
        out_specs=pl.BlockSpec((BN, D), lambda i: (i, 0)),
        out_shape=jax.ShapeDtypeStruct((N, D), jnp.float32),
    )(x, parts_a, parts_b, wnT, bn, gh, bh)


def kernel(x, poly_conn, poly_index, WQ, bQ, WK, bK, WE, Wc, bc, Wn, bn,
           g_h, b_h, g_e, b_e):
    dst = poly_index[0]
    src = poly_index[1]
    dst_a, dst_b = dst[:EH], dst[EH:]
    src_a, src_b = src[:EH], src[EH:]
    zeros_nd = jnp.zeros((N_PAD, D), jnp.float32)

    qall, kall = _tc_qk(x, WQ.T, bQ[None], WK.T, bK[None])
    qd_a, ks_a = _sc_gather(qall, kall, dst_a, src_a)
    conn_a, e_a = _tc_edge(poly_conn, qd_a, ks_a, WE.T, Wc.T, bc[None],
                           g_e[None], b_e[None], 0)
    qd_b, ks_b = _sc_gather(qall, kall, dst_b, src_b)
    parts_a = _sc_scatter(conn_a, dst_a, zeros_nd)
    conn_b, e = _tc_edge(poly_conn, qd_b, ks_b, WE.T, Wc.T, bc[None],
                         g_e[None], b_e[None], 1, e_a)
    parts_b = _sc_scatter(conn_b, dst_b, zeros_nd)
    h = _tc_node(x, parts_a, parts_b, Wn.T, bn[None], g_h[None], b_h[None])
    return (h, e)


# speedup vs baseline: 4.4607x; 1.0008x over previous
import functools

import jax
import jax.numpy as jnp
from jax import lax
from jax.experimental import pallas as pl
from jax.experimental.pallas import tpu as pltpu
from jax.experimental.pallas import tpu_sc as plsc

N = 10000
E = 320000
D = 128
EPS = 1e-5

NC = 2
NS = 16
NW = NC * NS
CHUNK = 128
GSETS = 3
SSETS = 2
N_PAD = 10240
RPT = N_PAD // NS

QSIZES = (81920, 81920, 81920, 74240)
QOFFS = (0, 81920, 163840, 245760)
QBLOCK = (2048, 2048, 2048, 2560)


def _vmesh():
    return plsc.VectorSubcoreMesh(core_axis_name="c", subcore_axis_name="s",
                                  num_cores=NC, num_subcores=NS)


def _ln_rows(v, g, b):
    mu = jnp.mean(v, axis=-1, keepdims=True)
    d = v - mu
    var = jnp.mean(d * d, axis=-1, keepdims=True)
    return g * (d * lax.rsqrt(var + EPS)) + b


def _tc_qk(x, wqT, bq, wkT, bk):
    BN = 2000

    def body(x_r, wq_r, bq_r, wk_r, bk_r, q_r, k_r):
        xb = x_r[...]
        q_r[...] = jnp.dot(xb, wq_r[...], preferred_element_type=jnp.float32) + bq_r[...]
        k_r[...] = jnp.dot(xb, wk_r[...], preferred_element_type=jnp.float32) + bk_r[...]

    return pl.pallas_call(
        body,
        grid=(N // BN,),
        in_specs=[pl.BlockSpec((BN, D), lambda i: (i, 0)),
                  pl.BlockSpec((D, D), lambda i: (0, 0)),
                  pl.BlockSpec((1, D), lambda i: (0, 0)),
                  pl.BlockSpec((D, D), lambda i: (0, 0)),
                  pl.BlockSpec((1, D), lambda i: (0, 0))],
        out_specs=[pl.BlockSpec((BN, D), lambda i: (i, 0)),
                   pl.BlockSpec((BN, D), lambda i: (i, 0))],
        out_shape=[jax.ShapeDtypeStruct((N, D), jnp.float32)] * 2,
    )(x, wqT, bq, wkT, bk)


def _sc_gather(qall, kall, dst_q, src_q, eh):
    epw = eh // NW
    gch = epw // CHUNK
    grem = epw - gch * CHUNK
    gp = gch // GSETS
    gtail = gch - gp * GSETS

    @functools.partial(
        pl.kernel, mesh=_vmesh(),
        out_type=(jax.ShapeDtypeStruct((eh, D), jnp.float32),
                  jax.ShapeDtypeStruct((eh, D), jnp.float32)),
        scratch_types=[pltpu.VMEM((epw,), jnp.int32),
                       pltpu.VMEM((epw,), jnp.int32)]
                      + [pltpu.VMEM((CHUNK, D), jnp.float32)] * (2 * GSETS)
                      + [pltpu.SemaphoreType.DMA] * (2 * GSETS),
    )
    def k(q_hbm, k_hbm, dst_hbm, src_hbm, qd_hbm, ks_hbm, dsti, srci, *bufs):
        rqs = bufs[0:GSETS]
        rks = bufs[GSETS:2 * GSETS]
        gsems = bufs[2 * GSETS:3 * GSETS]
        ssems = bufs[3 * GSETS:4 * GSETS]
        wid = lax.axis_index("s") * NC + lax.axis_index("c")
        eb = wid * epw
        pltpu.sync_copy(dst_hbm.at[pl.ds(eb, epw)], dsti)
        pltpu.sync_copy(src_hbm.at[pl.ds(eb, epw)], srci)

        def issue_gather(c, s):
            pltpu.async_copy(q_hbm.at[dsti.at[pl.ds(c * CHUNK, CHUNK)]],
                             rqs[s], gsems[s])
            pltpu.async_copy(k_hbm.at[srci.at[pl.ds(c * CHUNK, CHUNK)]],
                             rks[s], gsems[s])

        def wait_gather(s):
            pltpu.make_async_copy(q_hbm.at[dsti.at[pl.ds(0, CHUNK)]],
                                  rqs[s], gsems[s]).wait()
            pltpu.make_async_copy(k_hbm.at[srci.at[pl.ds(0, CHUNK)]],
                                  rks[s], gsems[s]).wait()

        def issue_store(c, s):
            pltpu.async_copy(rqs[s], qd_hbm.at[pl.ds(eb + c * CHUNK, CHUNK)],
                             ssems[s])
            pltpu.async_copy(rks[s], ks_hbm.at[pl.ds(eb + c * CHUNK, CHUNK)],
                             ssems[s])

        def wait_store(s):
            pltpu.make_async_copy(rqs[s], qd_hbm.at[pl.ds(eb, CHUNK)],
                                  ssems[s]).wait()
            pltpu.make_async_copy(rks[s], ks_hbm.at[pl.ds(eb, CHUNK)],
                                  ssems[s]).wait()

        for s in range(GSETS):
            issue_gather(s, s)

        def body(j, carry):
            c = GSETS * j
            for s in range(GSETS):
                wait_gather(s)
                issue_store(c + s, s)
            for s in range(GSETS):
                wait_store(s)

                @pl.when(c + s + GSETS < gch)
                def _():
                    issue_gather(c + s + GSETS, s)

            return carry

        lax.fori_loop(0, gp, body, 0)

        for t in range(gtail):
            c = gp * GSETS + t
            wait_gather(t)
            issue_store(c, t)
            wait_store(t)

        if grem > 0:
            pltpu.async_copy(q_hbm.at[dsti.at[pl.ds(gch * CHUNK, grem)]],
                             rqs[0].at[pl.ds(0, grem)], gsems[0])
            pltpu.async_copy(k_hbm.at[srci.at[pl.ds(gch * CHUNK, grem)]],
                             rks[0].at[pl.ds(0, grem)], gsems[0])
            pltpu.make_async_copy(q_hbm.at[dsti.at[pl.ds(0, grem)]],
                                  rqs[0].at[pl.ds(0, grem)], gsems[0]).wait()
            pltpu.make_async_copy(k_hbm.at[srci.at[pl.ds(0, grem)]],
                                  rks[0].at[pl.ds(0, grem)], gsems[0]).wait()
            pltpu.sync_copy(rqs[0].at[pl.ds(0, grem)],
                            qd_hbm.at[pl.ds(eb + gch * CHUNK, grem)])
            pltpu.sync_copy(rks[0].at[pl.ds(0, grem)],
                            ks_hbm.at[pl.ds(eb + gch * CHUNK, grem)])

    return k(qall, kall, dst_q, src_q)


def _tc_edge(pc, qd, ks, weT, wcT, bc, ge, be, qi, e_prev=None):
    eh = QSIZES[qi]
    be_blk = QBLOCK[qi]
    hoff = QOFFS[qi] // be_blk

    def body(pc_r, qd_r, ks_r, we_r, wc_r, bc_r, ge_r, be_r, *rest):
        conn_r, e_r = rest[-2], rest[-1]
        pcb = pc_r[...]
        eh_v = jnp.dot(pcb, we_r[...], preferred_element_type=jnp.float32)
        c1 = jnp.maximum(qd_r[...] + ks_r[...] + eh_v, 0.0)
        conn = jnp.dot(c1, wc_r[...], preferred_element_type=jnp.float32) + bc_r[...]
        conn_r[...] = conn
        e_r[...] = _ln_rows(pcb + conn, ge_r[...], be_r[...])

    in_specs = [pl.BlockSpec((be_blk, D), lambda i: (i + hoff, 0)),
                pl.BlockSpec((be_blk, D), lambda i: (i, 0)),
                pl.BlockSpec((be_blk, D), lambda i: (i, 0)),
                pl.BlockSpec((D, D), lambda i: (0, 0)),
                pl.BlockSpec((D, D), lambda i: (0, 0)),
                pl.BlockSpec((1, D), lambda i: (0, 0)),
                pl.BlockSpec((1, D), lambda i: (0, 0)),
                pl.BlockSpec((1, D), lambda i: (0, 0))]
    args = (pc, qd, ks, weT, wcT, bc, ge, be)
    aliases = {}
    if e_prev is not None:
        in_specs.append(pl.BlockSpec(memory_space=pl.ANY))
        args = args + (e_prev,)
        aliases = {8: 1}

    return pl.pallas_call(
        body,
        grid=(eh // be_blk,),
        in_specs=in_specs,
        out_specs=[pl.BlockSpec((be_blk, D), lambda i: (i, 0)),
                   pl.BlockSpec((be_blk, D), lambda i: (i + hoff, 0))],
        out_shape=[jax.ShapeDtypeStruct((eh, D), jnp.float32),
                   jax.ShapeDtypeStruct((E, D), jnp.float32)],
        input_output_aliases=aliases,
    )(*args)


def _sc_scatter(conn_q, dst_q, zeros_nd, eh):
    epw = eh // NW
    gch = epw // CHUNK
    grem = epw - gch * CHUNK
    sgp = gch // SSETS
    stail = gch - sgp * SSETS

    @functools.partial(
        pl.kernel, mesh=_vmesh(),
        out_type=jax.ShapeDtypeStruct((NC, N_PAD, D), jnp.float32),
        scratch_types=[pltpu.VMEM_SHARED((N_PAD, D), jnp.float32),
                       pltpu.VMEM((max(grem, 8),), jnp.int32)]
                      + [pltpu.VMEM((CHUNK,), jnp.int32)] * SSETS
                      + [pltpu.VMEM((CHUNK, D), jnp.float32)] * SSETS
                      + [pltpu.SemaphoreType.DMA] * (2 * SSETS),
    )
    def k(conn_hbm, dst_hbm, zero_hbm, out_hbm, acc_sh, idx_rem, *bufs):
        idxs = bufs[0:SSETS]
        rows = bufs[SSETS:2 * SSETS]
        lsems = bufs[2 * SSETS:3 * SSETS]
        asems = bufs[3 * SSETS:4 * SSETS]
        cid = lax.axis_index("c")
        sid = lax.axis_index("s")
        wid = sid * NC + cid
        eb = wid * epw
        pltpu.sync_copy(zero_hbm.at[pl.ds(sid * RPT, RPT)],
                        acc_sh.at[pl.ds(sid * RPT, RPT)])
        plsc.subcore_barrier()

        def issue_load(c, s):
            base = eb + c * CHUNK
            pltpu.async_copy(dst_hbm.at[pl.ds(base, CHUNK)], idxs[s], lsems[s])
            pltpu.async_copy(conn_hbm.at[pl.ds(base, CHUNK)], rows[s], lsems[s])

        def wait_load(s):
            pltpu.make_async_copy(dst_hbm.at[pl.ds(0, CHUNK)],
                                  idxs[s], lsems[s]).wait()
            pltpu.make_async_copy(conn_hbm.at[pl.ds(0, CHUNK)],
                                  rows[s], lsems[s]).wait()

        def issue_scatter(s):
            pltpu.async_copy(rows[s], acc_sh.at[idxs[s]], asems[s], add=True)

        def wait_scatter(s):
            pltpu.make_async_copy(rows[s], acc_sh.at[idxs[s]], asems[s]).wait()

        for s in range(SSETS):
            issue_load(s, s)

        def body(j, carry):
            c = SSETS * j
            for s in range(SSETS):
                wait_load(s)
                issue_scatter(s)
            for s in range(SSETS):
                wait_scatter(s)

                @pl.when(c + s + SSETS < gch)
                def _():
                    issue_load(c + s + SSETS, s)

            return carry

        lax.fori_loop(0, sgp, body, 0)

        for t in range(stail):
            wait_load(t)
            issue_scatter(t)
            wait_scatter(t)

        if grem > 0:
            rbase = eb + gch * CHUNK
            pltpu.sync_copy(dst_hbm.at[pl.ds(rbase, grem)],
                            idx_rem.at[pl.ds(0, grem)])
            pltpu.sync_copy(conn_hbm.at[pl.ds(rbase, grem)],
                            rows[0].at[pl.ds(0, grem)])
            pltpu.sync_copy(rows[0].at[pl.ds(0, grem)],
                            acc_sh.at[idx_rem.at[pl.ds(0, grem)]], add=True)

        plsc.subcore_barrier()
        pltpu.sync_copy(acc_sh.at[pl.ds(sid * RPT, RPT)],
                        out_hbm.at[cid, pl.ds(sid * RPT, RPT)])

    return k(conn_q, dst_q, zeros_nd)


def _tc_node(x, parts, wnT, bn, gh, bh):
    BN = 2000
    nq = len(parts)

    def body(x_r, *rest):
        p_rs = rest[:nq]
        wn_r, bn_r, gh_r, bh_r, h_r = rest[nq:]
        agg = p_rs[0][0] + p_rs[0][1]
        for p_r in p_rs[1:]:
            agg = agg + p_r[0] + p_r[1]
        h = x_r[...] + jnp.dot(agg, wn_r[...], preferred_element_type=jnp.float32) + bn_r[...]
        h_r[...] = _ln_rows(h, gh_r[...], bh_r[...])

    return pl.pallas_call(
        body,
        grid=(N // BN,),
        in_specs=[pl.BlockSpec((BN, D), lambda i: (i, 0))]
                 + [pl.BlockSpec((NC, BN, D), lambda i: (0, i, 0))] * nq
                 + [pl.BlockSpec((D, D), lambda i: (0, 0)),
                    pl.BlockSpec((1, D), lambda i: (0, 0)),
                    pl.BlockSpec((1, D), lambda i: (0, 0)),
                    pl.BlockSpec((1, D), lambda i: (0, 0))],
        out_specs=pl.BlockSpec((BN, D), lambda i: (i, 0)),
        out_shape=jax.ShapeDtypeStruct((N, D), jnp.float32),
    )(x, *parts, wnT, bn, gh, bh)


def kernel(x, poly_conn, poly_index, WQ, bQ, WK, bK, WE, Wc, bc, Wn, bn,
           g_h, b_h, g_e, b_e):
    dst = poly_index[0]
    src = poly_index[1]
    nq = len(QSIZES)
    dst_q = [lax.slice(dst, (QOFFS[i],), (QOFFS[i] + QSIZES[i],))
             for i in range(nq)]
    src_q = [lax.slice(src, (QOFFS[i],), (QOFFS[i] + QSIZES[i],))
             for i in range(nq)]
    zeros_nd = jnp.zeros((N_PAD, D), jnp.float32)
    weT, wcT = WE.T, Wc.T
    bc_, ge_, be_ = bc[None], g_e[None], b_e[None]

    qall, kall = _tc_qk(x, WQ.T, bQ[None], WK.T, bK[None])

    gathered = {0: _sc_gather(qall, kall, dst_q[0], src_q[0], QSIZES[0])}
    conns = {}
    parts = []
    e_cur = None
    for i in range(nq):
        qd_i, ks_i = gathered.pop(i)
        conn_i, e_cur = _tc_edge(poly_conn, qd_i, ks_i, weT, wcT, bc_,
                                 ge_, be_, i, e_cur)
        conns[i] = conn_i
        if i + 1 < nq:
            gathered[i + 1] = _sc_gather(qall, kall, dst_q[i + 1],
                                         src_q[i + 1], QSIZES[i + 1])
        if i - 1 >= 0:
            parts.append(_sc_scatter(conns.pop(i - 1), dst_q[i - 1],
                                     zeros_nd, QSIZES[i - 1]))
    parts.append(_sc_scatter(conns.pop(nq - 1), dst_q[nq - 1],
                             zeros_nd, QSIZES[nq - 1]))

    h = _tc_node(x, parts, Wn.T, bn[None], g_h[None], b_h[None])
    return (h, e_cur)
